# baseline (speedup 1.0000x reference)
import functools

import numpy as np
import jax
import jax.numpy as jnp
from jax import lax
from jax.experimental import pallas as pl
from jax.experimental.pallas import tpu as pltpu
from jax.experimental.pallas import tpu_sc as plsc

B = 16384
F = 26
VOCAB_PER_FIELD = 40000
TOTAL_VOCAB = F * VOCAB_PER_FIELD
D = 16
FD = F * D
H1 = 128
H2 = 64
EPS = 1e-5
THR = 1.0 / (F * 0.5)

SLOT = 32
FDP = SLOT * D
NIDX = B * SLOT
NW = 32
PER_W = NIDX // NW
GW = 128
NCH = PER_W // GW
TB = 1024


VBLK = 80000
VSUB = VBLK // 8


def _table_rows(table_t):
    def body(in_ref, out_ref):
        t = in_ref[...]
        col = jax.lax.broadcasted_iota(jnp.int32, (D, 8 * D), 1)
        row = jax.lax.broadcasted_iota(jnp.int32, (D, 8 * D), 0)
        acc = None
        for j in range(8):
            ej = (col == j * D + row).astype(jnp.float32)
            part = jax.lax.dot_general(
                t[:, j * VSUB:(j + 1) * VSUB], ej, (((0,), (0,)), ((), ())),
                preferred_element_type=jnp.float32)
            acc = part if acc is None else acc + part
        out_ref[...] = acc

    return pl.pallas_call(
        body,
        grid=(TOTAL_VOCAB // VBLK,),
        in_specs=[pl.BlockSpec((D, VBLK), lambda i: (0, i))],
        out_specs=pl.BlockSpec((VSUB, 8 * D), lambda i: (i, 0)),
        out_shape=jax.ShapeDtypeStruct((TOTAL_VOCAB // 8, 8 * D), jnp.float32),
        compiler_params=pltpu.CompilerParams(dimension_semantics=("parallel",)),
    )(table_t)


GRP = 8
NGRP = NCH // GRP


def _sc_gather(table, idx2):
    mesh = plsc.VectorSubcoreMesh(core_axis_name="c", subcore_axis_name="s")

    @functools.partial(
        pl.kernel,
        mesh=mesh,
        out_type=jax.ShapeDtypeStruct((NIDX, D), jnp.float32),
        scratch_types=[
            pltpu.VMEM((PER_W,), jnp.int32),
            pltpu.VMEM((2, GRP, GW, D), jnp.float32),
            pltpu.SemaphoreType.DMA((2,)),
            pltpu.SemaphoreType.DMA((2,)),
        ],
        compiler_params=pltpu.CompilerParams(use_tc_tiling_on_sc=False),
    )
    def gather_kernel(tbl_hbm, idx_hbm, out_hbm, idx_all, rows, gsem, ssem):
        wid = lax.axis_index("s") * 2 + lax.axis_index("c")
        base = wid * PER_W
        pltpu.sync_copy(idx_hbm.at[wid], idx_all)

        def gat(jw, bank, s):
            return pltpu.make_async_copy(
                tbl_hbm.at[idx_all.at[pl.ds(jw * GW, GW)]],
                rows.at[bank, s], gsem.at[bank])

        def sto(jw, bank, s):
            return pltpu.make_async_copy(
                rows.at[bank, s], out_hbm.at[pl.ds(base + jw * GW, GW)],
                ssem.at[bank])

        def do_group(g, bank, wait_stores):
            if wait_stores:
                for s in range(GRP):
                    sto(0, bank, s).wait()
            for s in range(GRP):
                gat(g * GRP + s, bank, s).start()
            for s in range(GRP):
                gat(g * GRP + s, bank, s).wait()
            for s in range(GRP):
                sto(g * GRP + s, bank, s).start()

        do_group(0, 0, False)
        do_group(1, 1, False)

        @pl.loop(2, NGRP, step=2)
        def _(g0):
            do_group(g0, 0, True)
            do_group(g0 + 1, 1, True)

        for s in range(GRP):
            sto(0, 0, s).wait()
        for s in range(GRP):
            sto(0, 1, s).wait()

    return gather_kernel(table, idx2)


def _ctrl_mm(flat, cw, cb):
    def body(flat_ref, cw_ref, cb_ref, h_ref):
        h_ref[...] = jnp.dot(flat_ref[...], cw_ref[...],
                             preferred_element_type=jnp.float32) + cb_ref[...]

    return pl.pallas_call(
        body,
        grid=(B // TB,),
        in_specs=[
            pl.BlockSpec((TB, FDP), lambda i: (i, 0)),
            pl.BlockSpec((FDP, F), lambda i: (0, 0)),
            pl.BlockSpec((1, F), lambda i: (0, 0)),
        ],
        out_specs=pl.BlockSpec((TB, F), lambda i: (i, 0)),
        out_shape=jax.ShapeDtypeStruct((B, F), jnp.float32),
        compiler_params=pltpu.CompilerParams(dimension_semantics=("parallel",)),
    )(flat, cw, cb)


def _ctrl_mask(h_in, cg, cbeta):
    def body(h_ref, cg_ref, cbeta_ref, wn_ref):
        h = h_ref[...]
        m = jnp.mean(h, axis=0, keepdims=True)
        v = jnp.mean(jnp.square(h - m), axis=0, keepdims=True)
        h = (h - m) * jax.lax.rsqrt(v + EPS) * cg_ref[...] + cbeta_ref[...]
        h = jnp.maximum(h, 0.0)
        hmax = jnp.max(h, axis=1, keepdims=True)
        e = jnp.exp(h - hmax)
        w = e / jnp.sum(e, axis=1, keepdims=True)
        mask = (w >= THR).astype(jnp.float32)
        wmax = jnp.max(w, axis=1, keepdims=True)
        lane = jax.lax.broadcasted_iota(jnp.int32, w.shape, 1)
        first = jnp.min(jnp.where(w == wmax, lane, F), axis=1, keepdims=True)
        mask = jnp.maximum(mask, (lane == first).astype(jnp.float32))
        wn = w * mask
        wn_ref[...] = wn / jnp.sum(wn, axis=1, keepdims=True)

    return pl.pallas_call(
        body,
        out_shape=jax.ShapeDtypeStruct((B, F), jnp.float32),
    )(h_in, cg, cbeta)


def _mid(flat, wn, expand, w1, b1):
    def body(flat_ref, wn_ref, e_ref, w1_ref, b1_ref, z1_ref):
        wexp = jnp.dot(wn_ref[...], e_ref[...], preferred_element_type=jnp.float32)
        xw = flat_ref[...] * wexp
        z1_ref[...] = jnp.dot(xw, w1_ref[...], preferred_element_type=jnp.float32) + b1_ref[...]

    return pl.pallas_call(
        body,
        grid=(B // TB,),
        in_specs=[
            pl.BlockSpec((TB, FDP), lambda i: (i, 0)),
            pl.BlockSpec((TB, F), lambda i: (i, 0)),
            pl.BlockSpec((F, FDP), lambda i: (0, 0)),
            pl.BlockSpec((FDP, H1), lambda i: (0, 0)),
            pl.BlockSpec((1, H1), lambda i: (0, 0)),
        ],
        out_specs=pl.BlockSpec((TB, H1), lambda i: (i, 0)),
        out_shape=jax.ShapeDtypeStruct((B, H1), jnp.float32),
        compiler_params=pltpu.CompilerParams(dimension_semantics=("parallel",)),
    )(flat, wn, expand, w1, b1)


def _tail(z1, g1, beta1, w2, b2, g2, beta2, wo, bo):
    def body(z1_ref, g1_ref, beta1_ref, w2_ref, b2_ref, g2_ref, beta2_ref,
             wo_ref, bo_ref, out_ref):
        z = z1_ref[...]
        m = jnp.mean(z, axis=0, keepdims=True)
        v = jnp.mean(jnp.square(z - m), axis=0, keepdims=True)
        z = jnp.maximum((z - m) * jax.lax.rsqrt(v + EPS) * g1_ref[...] + beta1_ref[...], 0.0)
        z2 = jnp.dot(z, w2_ref[...], preferred_element_type=jnp.float32) + b2_ref[...]
        m2 = jnp.mean(z2, axis=0, keepdims=True)
        v2 = jnp.mean(jnp.square(z2 - m2), axis=0, keepdims=True)
        z2 = jnp.maximum((z2 - m2) * jax.lax.rsqrt(v2 + EPS) * g2_ref[...] + beta2_ref[...], 0.0)
        o = jnp.dot(z2, wo_ref[...], preferred_element_type=jnp.float32) + bo_ref[...]
        out_ref[...] = jax.nn.sigmoid(o)

    return pl.pallas_call(
        body,
        out_shape=jax.ShapeDtypeStruct((B, 1), jnp.float32),
    )(z1, g1, beta1, w2, b2, g2, beta2, wo, bo)


def kernel(x, table, cw, cb, cg, cbeta, w1, b1, g1, beta1, w2, b2, g2, beta2, wo, bo):
    offsets = (jnp.arange(F, dtype=jnp.int32) * VOCAB_PER_FIELD)[None, :]
    v = jnp.pad(x + offsets, ((0, 0), (0, SLOT - F)))
    i_blk = v // VBLK
    rem = v - i_blk * VBLK
    j_sub = rem // VSUB
    p_off = rem - j_sub * VSUB
    q = 8 * (i_blk * VSUB + p_off) + j_sub
    idx = q.reshape(NW, PER_W)
    tbl_rows = _table_rows(table.T).reshape(TOTAL_VOCAB, D)
    flat = _sc_gather(tbl_rows, idx).reshape(B, FDP)
    cwp = jnp.pad(cw, ((0, FDP - FD), (0, 0)))
    w1p = jnp.pad(w1, ((0, FDP - FD), (0, 0)))
    h = _ctrl_mm(flat, cwp, cb.reshape(1, F))
    wn = _ctrl_mask(h, cg.reshape(1, F), cbeta.reshape(1, F))
    expand = jnp.asarray(np.pad(
        np.kron(np.eye(F, dtype=np.float32), np.ones((1, D), np.float32)),
        ((0, 0), (0, FDP - FD))))
    z1 = _mid(flat, wn, expand, w1p, b1.reshape(1, H1))
    return _tail(z1, g1.reshape(1, H1), beta1.reshape(1, H1), w2, b2.reshape(1, H2),
                 g2.reshape(1, H2), beta2.reshape(1, H2), wo, bo.reshape(1, 1))


# speedup vs baseline: 2.2147x; 2.2147x over previous
import functools

import numpy as np
import jax
import jax.numpy as jnp
from jax import lax
from jax.experimental import pallas as pl
from jax.experimental.pallas import tpu as pltpu
from jax.experimental.pallas import tpu_sc as plsc

B = 16384
F = 26
VOCAB_PER_FIELD = 40000
TOTAL_VOCAB = F * VOCAB_PER_FIELD
D = 16
FD = F * D
H1 = 128
H2 = 64
EPS = 1e-5
THR = 1.0 / (F * 0.5)

SLOT = 32
FDP = SLOT * D
NIDX = B * SLOT
NW = 32
PER_W = NIDX // NW
GW = 128
NCH = PER_W // GW
TB = 1024


VBLK = 80000
VSUB = VBLK // 8


def _table_rows(table_t):
    def body(in_ref, out_ref):
        t = in_ref[...]
        col = jax.lax.broadcasted_iota(jnp.int32, (D, 8 * D), 1)
        row = jax.lax.broadcasted_iota(jnp.int32, (D, 8 * D), 0)
        acc = None
        for j in range(8):
            ej = (col == j * D + row).astype(jnp.float32)
            part = jax.lax.dot_general(
                t[:, j * VSUB:(j + 1) * VSUB], ej, (((0,), (0,)), ((), ())),
                preferred_element_type=jnp.float32)
            acc = part if acc is None else acc + part
        out_ref[...] = acc

    return pl.pallas_call(
        body,
        grid=(TOTAL_VOCAB // VBLK,),
        in_specs=[pl.BlockSpec((D, VBLK), lambda i: (0, i))],
        out_specs=pl.BlockSpec((VSUB, 8 * D), lambda i: (i, 0)),
        out_shape=jax.ShapeDtypeStruct((TOTAL_VOCAB // 8, 8 * D), jnp.float32),
        compiler_params=pltpu.CompilerParams(dimension_semantics=("parallel",)),
    )(table_t)


GRP = 8
NGRP = NCH // GRP


def _sc_gather(table, idx2):
    mesh = plsc.VectorSubcoreMesh(core_axis_name="c", subcore_axis_name="s")

    @functools.partial(
        pl.kernel,
        mesh=mesh,
        out_type=jax.ShapeDtypeStruct((NIDX, D), jnp.float32),
        scratch_types=[
            pltpu.VMEM((PER_W,), jnp.int32),
            pltpu.VMEM((2, GRP, GW, D), jnp.float32),
            pltpu.SemaphoreType.DMA((2,)),
            pltpu.SemaphoreType.DMA((2,)),
        ],
        compiler_params=pltpu.CompilerParams(use_tc_tiling_on_sc=False),
    )
    def gather_kernel(tbl_hbm, idx_hbm, out_hbm, idx_all, rows, gsem, ssem):
        wid = lax.axis_index("s") * 2 + lax.axis_index("c")
        base = wid * PER_W
        pltpu.sync_copy(idx_hbm.at[wid], idx_all)

        def gat(jw, bank, s):
            return pltpu.make_async_copy(
                tbl_hbm.at[idx_all.at[pl.ds(jw * GW, GW)]],
                rows.at[bank, s], gsem.at[bank])

        def sto(jw, bank, s):
            return pltpu.make_async_copy(
                rows.at[bank, s], out_hbm.at[pl.ds(base + jw * GW, GW)],
                ssem.at[bank])

        def do_group(g, bank, wait_stores):
            if wait_stores:
                for s in range(GRP):
                    sto(0, bank, s).wait()
            for s in range(GRP):
                gat(g * GRP + s, bank, s).start()
            for s in range(GRP):
                gat(g * GRP + s, bank, s).wait()
            for s in range(GRP):
                sto(g * GRP + s, bank, s).start()

        do_group(0, 0, False)
        do_group(1, 1, False)

        @pl.loop(2, NGRP, step=2)
        def _(g0):
            do_group(g0, 0, True)
            do_group(g0 + 1, 1, True)

        for s in range(GRP):
            sto(0, 0, s).wait()
        for s in range(GRP):
            sto(0, 1, s).wait()

    return gather_kernel(table, idx2)


def _ctrl_mm(flat, cw, cb):
    def body(flat_ref, cw_ref, cb_ref, h_ref):
        h_ref[...] = jnp.dot(flat_ref[...], cw_ref[...],
                             preferred_element_type=jnp.float32) + cb_ref[...]

    return pl.pallas_call(
        body,
        grid=(B // TB,),
        in_specs=[
            pl.BlockSpec((TB, FDP), lambda i: (i, 0)),
            pl.BlockSpec((FDP, F), lambda i: (0, 0)),
            pl.BlockSpec((1, F), lambda i: (0, 0)),
        ],
        out_specs=pl.BlockSpec((TB, F), lambda i: (i, 0)),
        out_shape=jax.ShapeDtypeStruct((B, F), jnp.float32),
        compiler_params=pltpu.CompilerParams(dimension_semantics=("parallel",)),
    )(flat, cw, cb)


def _ctrl_mask(h_in, cg, cbeta):
    def body(h_ref, cg_ref, cbeta_ref, wn_ref):
        h = h_ref[...]
        m = jnp.mean(h, axis=0, keepdims=True)
        v = jnp.mean(jnp.square(h - m), axis=0, keepdims=True)
        h = (h - m) * jax.lax.rsqrt(v + EPS) * cg_ref[...] + cbeta_ref[...]
        h = jnp.maximum(h, 0.0)
        hmax = jnp.max(h, axis=1, keepdims=True)
        e = jnp.exp(h - hmax)
        w = e / jnp.sum(e, axis=1, keepdims=True)
        mask = (w >= THR).astype(jnp.float32)
        wmax = jnp.max(w, axis=1, keepdims=True)
        lane = jax.lax.broadcasted_iota(jnp.int32, w.shape, 1)
        first = jnp.min(jnp.where(w == wmax, lane, F), axis=1, keepdims=True)
        mask = jnp.maximum(mask, (lane == first).astype(jnp.float32))
        wn = w * mask
        wn_ref[...] = wn / jnp.sum(wn, axis=1, keepdims=True)

    return pl.pallas_call(
        body,
        out_shape=jax.ShapeDtypeStruct((B, F), jnp.float32),
    )(h_in, cg, cbeta)


def _mid(flat, wn, expand, w1, b1):
    def body(flat_ref, wn_ref, e_ref, w1_ref, b1_ref, z1_ref):
        wexp = jnp.dot(wn_ref[...], e_ref[...], preferred_element_type=jnp.float32)
        xw = flat_ref[...] * wexp
        z1_ref[...] = jnp.dot(xw, w1_ref[...], preferred_element_type=jnp.float32) + b1_ref[...]

    return pl.pallas_call(
        body,
        grid=(B // TB,),
        in_specs=[
            pl.BlockSpec((TB, FDP), lambda i: (i, 0)),
            pl.BlockSpec((TB, F), lambda i: (i, 0)),
            pl.BlockSpec((F, FDP), lambda i: (0, 0)),
            pl.BlockSpec((FDP, H1), lambda i: (0, 0)),
            pl.BlockSpec((1, H1), lambda i: (0, 0)),
        ],
        out_specs=pl.BlockSpec((TB, H1), lambda i: (i, 0)),
        out_shape=jax.ShapeDtypeStruct((B, H1), jnp.float32),
        compiler_params=pltpu.CompilerParams(dimension_semantics=("parallel",)),
    )(flat, wn, expand, w1, b1)


def _tail(z1, g1, beta1, w2, b2, g2, beta2, wo, bo):
    def body(z1_ref, g1_ref, beta1_ref, w2_ref, b2_ref, g2_ref, beta2_ref,
             wo_ref, bo_ref, out_ref):
        z = z1_ref[...]
        m = jnp.mean(z, axis=0, keepdims=True)
        v = jnp.mean(jnp.square(z - m), axis=0, keepdims=True)
        z = jnp.maximum((z - m) * jax.lax.rsqrt(v + EPS) * g1_ref[...] + beta1_ref[...], 0.0)
        z2 = jnp.dot(z, w2_ref[...], preferred_element_type=jnp.float32) + b2_ref[...]
        m2 = jnp.mean(z2, axis=0, keepdims=True)
        v2 = jnp.mean(jnp.square(z2 - m2), axis=0, keepdims=True)
        z2 = jnp.maximum((z2 - m2) * jax.lax.rsqrt(v2 + EPS) * g2_ref[...] + beta2_ref[...], 0.0)
        o = jnp.dot(z2, wo_ref[...], preferred_element_type=jnp.float32) + bo_ref[...]
        out_ref[...] = jax.nn.sigmoid(o)

    return pl.pallas_call(
        body,
        out_shape=jax.ShapeDtypeStruct((B, 1), jnp.float32),
    )(z1, g1, beta1, w2, b2, g2, beta2, wo, bo)


def kernel(x, table, cw, cb, cg, cbeta, w1, b1, g1, beta1, w2, b2, g2, beta2, wo, bo):
    offsets = (jnp.arange(F, dtype=jnp.int32) * VOCAB_PER_FIELD)[None, :]
    dummy = (jax.lax.broadcasted_iota(jnp.int32, (B, SLOT - F), 0) * (SLOT - F)
             + jax.lax.broadcasted_iota(jnp.int32, (B, SLOT - F), 1)) % TOTAL_VOCAB
    v = jnp.concatenate([x + offsets, dummy], axis=1)
    i_blk = v // VBLK
    rem = v - i_blk * VBLK
    j_sub = rem // VSUB
    p_off = rem - j_sub * VSUB
    q = 8 * (i_blk * VSUB + p_off) + j_sub
    idx = q.reshape(NW, PER_W)
    tbl_rows = _table_rows(table.T).reshape(TOTAL_VOCAB, D)
    flat = _sc_gather(tbl_rows, idx).reshape(B, FDP)
    cwp = jnp.pad(cw, ((0, FDP - FD), (0, 0)))
    w1p = jnp.pad(w1, ((0, FDP - FD), (0, 0)))
    h = _ctrl_mm(flat, cwp, cb.reshape(1, F))
    wn = _ctrl_mask(h, cg.reshape(1, F), cbeta.reshape(1, F))
    expand = jnp.asarray(np.pad(
        np.kron(np.eye(F, dtype=np.float32), np.ones((1, D), np.float32)),
        ((0, 0), (0, FDP - FD))))
    z1 = _mid(flat, wn, expand, w1p, b1.reshape(1, H1))
    return _tail(z1, g1.reshape(1, H1), beta1.reshape(1, H1), w2, b2.reshape(1, H2),
                 g2.reshape(1, H2), beta2.reshape(1, H2), wo, bo.reshape(1, 1))


# speedup vs baseline: 2.2491x; 1.0155x over previous
import functools

import numpy as np
import jax
import jax.numpy as jnp
from jax import lax
from jax.experimental import pallas as pl
from jax.experimental.pallas import tpu as pltpu
from jax.experimental.pallas import tpu_sc as plsc

B = 16384
F = 26
VOCAB_PER_FIELD = 40000
TOTAL_VOCAB = F * VOCAB_PER_FIELD
D = 16
FD = F * D
H1 = 128
H2 = 64
EPS = 1e-5
THR = 1.0 / (F * 0.5)

SLOT = 32
FDP = SLOT * D
NIDX = B * SLOT
NW = 32
PER_W = NIDX // NW
GW = 128
NCH = PER_W // GW
TB = 1024


VBLK = 80000
VSUB = VBLK // 8


def _table_rows(table_t):
    def body(in_ref, out_ref):
        t = in_ref[...]
        col = jax.lax.broadcasted_iota(jnp.int32, (D, 8 * D), 1)
        row = jax.lax.broadcasted_iota(jnp.int32, (D, 8 * D), 0)
        acc = None
        for j in range(8):
            ej = (col == j * D + row).astype(jnp.float32)
            part = jax.lax.dot_general(
                t[:, j * VSUB:(j + 1) * VSUB], ej, (((0,), (0,)), ((), ())),
                preferred_element_type=jnp.float32)
            acc = part if acc is None else acc + part
        out_ref[...] = acc

    return pl.pallas_call(
        body,
        grid=(TOTAL_VOCAB // VBLK,),
        in_specs=[pl.BlockSpec((D, VBLK), lambda i: (0, i))],
        out_specs=pl.BlockSpec((VSUB, 8 * D), lambda i: (i, 0)),
        out_shape=jax.ShapeDtypeStruct((TOTAL_VOCAB // 8, 8 * D), jnp.float32),
        compiler_params=pltpu.CompilerParams(dimension_semantics=("parallel",)),
    )(table_t)


GRP = 16
NGRP = NCH // GRP


def _sc_gather(table, idx2):
    mesh = plsc.VectorSubcoreMesh(core_axis_name="c", subcore_axis_name="s")

    @functools.partial(
        pl.kernel,
        mesh=mesh,
        out_type=jax.ShapeDtypeStruct((NIDX, D), jnp.float32),
        scratch_types=[
            pltpu.VMEM((PER_W,), jnp.int32),
            pltpu.VMEM((2, GRP, GW, D), jnp.float32),
            pltpu.SemaphoreType.DMA((2,)),
            pltpu.SemaphoreType.DMA((2,)),
        ],
        compiler_params=pltpu.CompilerParams(use_tc_tiling_on_sc=False),
    )
    def gather_kernel(tbl_hbm, idx_hbm, out_hbm, idx_all, rows, gsem, ssem):
        wid = lax.axis_index("s") * 2 + lax.axis_index("c")
        base = wid * PER_W
        pltpu.sync_copy(idx_hbm.at[wid], idx_all)

        def gat(jw, bank, s):
            return pltpu.make_async_copy(
                tbl_hbm.at[idx_all.at[pl.ds(jw * GW, GW)]],
                rows.at[bank, s], gsem.at[bank])

        def sto(jw, bank, s):
            return pltpu.make_async_copy(
                rows.at[bank, s], out_hbm.at[pl.ds(base + jw * GW, GW)],
                ssem.at[bank])

        def do_group(g, bank, wait_stores):
            if wait_stores:
                for s in range(GRP):
                    sto(0, bank, s).wait()
            for s in range(GRP):
                gat(g * GRP + s, bank, s).start()
            for s in range(GRP):
                gat(g * GRP + s, bank, s).wait()
            for s in range(GRP):
                sto(g * GRP + s, bank, s).start()

        do_group(0, 0, False)
        do_group(1, 1, False)

        @pl.loop(2, NGRP, step=2)
        def _(g0):
            do_group(g0, 0, True)
            do_group(g0 + 1, 1, True)

        for s in range(GRP):
            sto(0, 0, s).wait()
        for s in range(GRP):
            sto(0, 1, s).wait()

    return gather_kernel(table, idx2)


def _ctrl_mm(flat, cw, cb):
    def body(flat_ref, cw_ref, cb_ref, h_ref):
        h_ref[...] = jnp.dot(flat_ref[...], cw_ref[...],
                             preferred_element_type=jnp.float32) + cb_ref[...]

    return pl.pallas_call(
        body,
        grid=(B // TB,),
        in_specs=[
            pl.BlockSpec((TB, FDP), lambda i: (i, 0)),
            pl.BlockSpec((FDP, F), lambda i: (0, 0)),
            pl.BlockSpec((1, F), lambda i: (0, 0)),
        ],
        out_specs=pl.BlockSpec((TB, F), lambda i: (i, 0)),
        out_shape=jax.ShapeDtypeStruct((B, F), jnp.float32),
        compiler_params=pltpu.CompilerParams(dimension_semantics=("parallel",)),
    )(flat, cw, cb)


def _ctrl_mask(h_in, cg, cbeta):
    def body(h_ref, cg_ref, cbeta_ref, wn_ref):
        h = h_ref[...]
        m = jnp.mean(h, axis=0, keepdims=True)
        v = jnp.mean(jnp.square(h - m), axis=0, keepdims=True)
        h = (h - m) * jax.lax.rsqrt(v + EPS) * cg_ref[...] + cbeta_ref[...]
        h = jnp.maximum(h, 0.0)
        hmax = jnp.max(h, axis=1, keepdims=True)
        e = jnp.exp(h - hmax)
        w = e / jnp.sum(e, axis=1, keepdims=True)
        mask = (w >= THR).astype(jnp.float32)
        wmax = jnp.max(w, axis=1, keepdims=True)
        lane = jax.lax.broadcasted_iota(jnp.int32, w.shape, 1)
        first = jnp.min(jnp.where(w == wmax, lane, F), axis=1, keepdims=True)
        mask = jnp.maximum(mask, (lane == first).astype(jnp.float32))
        wn = w * mask
        wn_ref[...] = wn / jnp.sum(wn, axis=1, keepdims=True)

    return pl.pallas_call(
        body,
        out_shape=jax.ShapeDtypeStruct((B, F), jnp.float32),
    )(h_in, cg, cbeta)


def _mid(flat, wn, expand, w1, b1):
    def body(flat_ref, wn_ref, e_ref, w1_ref, b1_ref, z1_ref):
        wexp = jnp.dot(wn_ref[...], e_ref[...], preferred_element_type=jnp.float32)
        xw = flat_ref[...] * wexp
        z1_ref[...] = jnp.dot(xw, w1_ref[...], preferred_element_type=jnp.float32) + b1_ref[...]

    return pl.pallas_call(
        body,
        grid=(B // TB,),
        in_specs=[
            pl.BlockSpec((TB, FDP), lambda i: (i, 0)),
            pl.BlockSpec((TB, F), lambda i: (i, 0)),
            pl.BlockSpec((F, FDP), lambda i: (0, 0)),
            pl.BlockSpec((FDP, H1), lambda i: (0, 0)),
            pl.BlockSpec((1, H1), lambda i: (0, 0)),
        ],
        out_specs=pl.BlockSpec((TB, H1), lambda i: (i, 0)),
        out_shape=jax.ShapeDtypeStruct((B, H1), jnp.float32),
        compiler_params=pltpu.CompilerParams(dimension_semantics=("parallel",)),
    )(flat, wn, expand, w1, b1)


def _tail(z1, g1, beta1, w2, b2, g2, beta2, wo, bo):
    def body(z1_ref, g1_ref, beta1_ref, w2_ref, b2_ref, g2_ref, beta2_ref,
             wo_ref, bo_ref, out_ref):
        z = z1_ref[...]
        m = jnp.mean(z, axis=0, keepdims=True)
        v = jnp.mean(jnp.square(z - m), axis=0, keepdims=True)
        z = jnp.maximum((z - m) * jax.lax.rsqrt(v + EPS) * g1_ref[...] + beta1_ref[...], 0.0)
        z2 = jnp.dot(z, w2_ref[...], preferred_element_type=jnp.float32) + b2_ref[...]
        m2 = jnp.mean(z2, axis=0, keepdims=True)
        v2 = jnp.mean(jnp.square(z2 - m2), axis=0, keepdims=True)
        z2 = jnp.maximum((z2 - m2) * jax.lax.rsqrt(v2 + EPS) * g2_ref[...] + beta2_ref[...], 0.0)
        o = jnp.dot(z2, wo_ref[...], preferred_element_type=jnp.float32) + bo_ref[...]
        out_ref[...] = jax.nn.sigmoid(o)

    return pl.pallas_call(
        body,
        out_shape=jax.ShapeDtypeStruct((B, 1), jnp.float32),
    )(z1, g1, beta1, w2, b2, g2, beta2, wo, bo)


def kernel(x, table, cw, cb, cg, cbeta, w1, b1, g1, beta1, w2, b2, g2, beta2, wo, bo):
    offsets = (jnp.arange(F, dtype=jnp.int32) * VOCAB_PER_FIELD)[None, :]
    dummy = (jax.lax.broadcasted_iota(jnp.int32, (B, SLOT - F), 0) * (SLOT - F)
             + jax.lax.broadcasted_iota(jnp.int32, (B, SLOT - F), 1)) % TOTAL_VOCAB
    v = jnp.concatenate([x + offsets, dummy], axis=1)
    i_blk = v // VBLK
    rem = v - i_blk * VBLK
    j_sub = rem // VSUB
    p_off = rem - j_sub * VSUB
    q = 8 * (i_blk * VSUB + p_off) + j_sub
    idx = q.reshape(NW, PER_W)
    tbl_rows = _table_rows(table.T).reshape(TOTAL_VOCAB, D)
    flat = _sc_gather(tbl_rows, idx).reshape(B, FDP)
    cwp = jnp.pad(cw, ((0, FDP - FD), (0, 0)))
    w1p = jnp.pad(w1, ((0, FDP - FD), (0, 0)))
    h = _ctrl_mm(flat, cwp, cb.reshape(1, F))
    wn = _ctrl_mask(h, cg.reshape(1, F), cbeta.reshape(1, F))
    expand = jnp.asarray(np.pad(
        np.kron(np.eye(F, dtype=np.float32), np.ones((1, D), np.float32)),
        ((0, 0), (0, FDP - FD))))
    z1 = _mid(flat, wn, expand, w1p, b1.reshape(1, H1))
    return _tail(z1, g1.reshape(1, H1), beta1.reshape(1, H1), w2, b2.reshape(1, H2),
                 g2.reshape(1, H2), beta2.reshape(1, H2), wo, bo.reshape(1, 1))
